# baseline (speedup 1.0000x reference)
import functools

import jax
import jax.numpy as jnp
from jax import lax
from jax.experimental import pallas as pl
from jax.experimental.pallas import tpu as pltpu
from jax.experimental.pallas import tpu_sc as plsc

N_NODES = 1000000
BATCH = 16384
NUM_CORES = 2
NUM_SUBCORES = 16
NW = NUM_CORES * NUM_SUBCORES
B_PER_W = BATCH // NW
STAGE = ((N_NODES // NUM_SUBCORES) // 128) * 128
TAIL0 = NUM_SUBCORES * STAGE
TAIL = N_NODES - TAIL0

_mesh = plsc.VectorSubcoreMesh(core_axis_name="c", subcore_axis_name="s")


@functools.partial(
    pl.kernel,
    mesh=_mesh,
    out_type=[jax.ShapeDtypeStruct((BATCH,), jnp.float32)] * 3,
    scratch_types=[
        pltpu.VMEM_SHARED((N_NODES,), jnp.float32),
        pltpu.VMEM((B_PER_W,), jnp.int32),
        pltpu.VMEM((B_PER_W,), jnp.float32),
        pltpu.VMEM((TAIL,), jnp.float32),
        pltpu.SemaphoreType.DMA,
        pltpu.SemaphoreType.DMA,
    ],
)
def _gather3(idx_hbm, mu_hbm, sg_hbm, et_hbm, out_mu, out_sg, out_et,
             table_sp, idx_v, buf_v, tail_v, stage_sem, sem):
    sid = lax.axis_index("s")
    wid = sid * NUM_CORES + lax.axis_index("c")
    base = wid * B_PER_W
    row0 = sid * STAGE
    pltpu.sync_copy(idx_hbm.at[pl.ds(base, B_PER_W)], idx_v)

    for tbl, out in ((mu_hbm, out_mu), (sg_hbm, out_sg), (et_hbm, out_et)):
        stage = pltpu.async_copy(
            tbl.at[pl.ds(row0, STAGE)], table_sp.at[pl.ds(row0, STAGE)],
            stage_sem)

        @pl.when(sid == NUM_SUBCORES - 1)
        def _():
            pltpu.sync_copy(tbl.at[pl.ds(TAIL0, TAIL)], tail_v)
            pltpu.sync_copy(tail_v, table_sp.at[pl.ds(TAIL0, TAIL)])

        stage.wait()
        plsc.subcore_barrier()
        pltpu.async_copy(table_sp.at[idx_v], buf_v, sem).wait()
        pltpu.sync_copy(buf_v, out.at[pl.ds(base, B_PER_W)])
        plsc.subcore_barrier()


def kernel(indices, mu_w, sigma_w, eta_w):
    mu, sg, et = _gather3(indices.astype(jnp.int32),
                          mu_w.reshape(-1), sigma_w.reshape(-1),
                          eta_w.reshape(-1))
    return jnp.stack([mu, sg, et], axis=-1)


# speedup vs baseline: 1.0950x; 1.0950x over previous
import functools

import jax
import jax.numpy as jnp
from jax import lax
from jax.experimental import pallas as pl
from jax.experimental.pallas import tpu as pltpu
from jax.experimental.pallas import tpu_sc as plsc

N_NODES = 1000000
BATCH = 16384
NUM_CORES = 2
NUM_SUBCORES = 16
NW = NUM_CORES * NUM_SUBCORES
B_PER_W = BATCH // NW
CHUNK = 128
NCHUNK = B_PER_W // CHUNK

_mesh = plsc.VectorSubcoreMesh(core_axis_name="c", subcore_axis_name="s")


@functools.partial(
    pl.kernel,
    mesh=_mesh,
    out_type=[jax.ShapeDtypeStruct((BATCH,), jnp.float32)] * 3,
    scratch_types=[
        pltpu.VMEM((B_PER_W,), jnp.int32),
        pltpu.VMEM((B_PER_W,), jnp.float32),
        pltpu.VMEM((B_PER_W,), jnp.float32),
        pltpu.VMEM((B_PER_W,), jnp.float32),
        pltpu.SemaphoreType.DMA,
    ],
)
def _gather3(idx_hbm, mu_hbm, sg_hbm, et_hbm, out_mu, out_sg, out_et,
             idx_v, buf_mu, buf_sg, buf_et, sem):
    wid = lax.axis_index("s") * NUM_CORES + lax.axis_index("c")
    base = wid * B_PER_W
    pltpu.sync_copy(idx_hbm.at[pl.ds(base, B_PER_W)], idx_v)
    copies = [
        pltpu.async_copy(tbl.at[idx_v.at[pl.ds(j * CHUNK, CHUNK)]],
                         buf.at[pl.ds(j * CHUNK, CHUNK)], sem)
        for tbl, buf in ((mu_hbm, buf_mu), (sg_hbm, buf_sg),
                         (et_hbm, buf_et))
        for j in range(NCHUNK)
    ]
    for c in copies:
        c.wait()
    pltpu.sync_copy(buf_mu, out_mu.at[pl.ds(base, B_PER_W)])
    pltpu.sync_copy(buf_sg, out_sg.at[pl.ds(base, B_PER_W)])
    pltpu.sync_copy(buf_et, out_et.at[pl.ds(base, B_PER_W)])


def kernel(indices, mu_w, sigma_w, eta_w):
    mu, sg, et = _gather3(indices.astype(jnp.int32),
                          mu_w.reshape(-1), sigma_w.reshape(-1),
                          eta_w.reshape(-1))
    return jnp.stack([mu, sg, et], axis=-1)


# speedup vs baseline: 1.0953x; 1.0003x over previous
import functools

import jax
import jax.numpy as jnp
from jax import lax
from jax.experimental import pallas as pl
from jax.experimental.pallas import tpu as pltpu
from jax.experimental.pallas import tpu_sc as plsc

N_NODES = 1000000
BATCH = 16384
NUM_CORES = 2
NUM_SUBCORES = 16
NW = NUM_CORES * NUM_SUBCORES
B_PER_W = BATCH // NW
CHUNK = 128
NCHUNK = B_PER_W // CHUNK

_mesh = plsc.VectorSubcoreMesh(core_axis_name="c", subcore_axis_name="s")


@functools.partial(
    pl.kernel,
    mesh=_mesh,
    out_type=[jax.ShapeDtypeStruct((BATCH,), jnp.float32)] * 3,
    scratch_types=[
        pltpu.VMEM((B_PER_W,), jnp.int32),
        pltpu.VMEM((B_PER_W,), jnp.float32),
        pltpu.VMEM((B_PER_W,), jnp.float32),
        pltpu.VMEM((B_PER_W,), jnp.float32),
        pltpu.SemaphoreType.DMA,
    ],
)
def _gather3(idx_hbm, mu_hbm, sg_hbm, et_hbm, out_mu, out_sg, out_et,
             idx_v, buf_mu, buf_sg, buf_et, sem):
    wid = lax.axis_index("s") * NUM_CORES + lax.axis_index("c")
    base = wid * B_PER_W
    pltpu.sync_copy(idx_hbm.at[pl.ds(base, B_PER_W)], idx_v)
    copies = [
        pltpu.async_copy(tbl.at[idx_v.at[pl.ds(j * CHUNK, CHUNK)]],
                         buf.at[pl.ds(j * CHUNK, CHUNK)], sem)
        for tbl, buf in ((mu_hbm, buf_mu), (sg_hbm, buf_sg),
                         (et_hbm, buf_et))
        for j in range(NCHUNK)
    ]
    for c in copies:
        c.wait()
    pltpu.sync_copy(buf_mu, out_mu.at[pl.ds(base, B_PER_W)])
    pltpu.sync_copy(buf_sg, out_sg.at[pl.ds(base, B_PER_W)])
    pltpu.sync_copy(buf_et, out_et.at[pl.ds(base, B_PER_W)])


def kernel(indices, mu_w, sigma_w, eta_w):
    mu, sg, et = _gather3(indices.astype(jnp.int32),
                          mu_w[:, 0], sigma_w[:, 0], eta_w[:, 0])
    return jnp.stack([mu, sg, et], axis=-1)
